# baseline (speedup 1.0000x reference)
import functools

import jax
import jax.numpy as jnp
import numpy as np
from jax import lax
from jax.experimental import layout as jax_layout
from jax.experimental import pallas as pl
from jax.experimental.pallas import tpu as pltpu
from jax.experimental.pallas import tpu_sc as plsc


def _edge_index_kernel(bsz, num_jobs, M, num_ops, conj_ref, psel_ref, pt_ref,
                       ei_ref):
    g = pl.program_id(0)
    P2 = num_jobs * (num_jobs - 1)
    E = (M + 1) * P2

    r0 = jax.lax.broadcasted_iota(jnp.int32, (num_ops, num_ops), 0)
    c0 = jax.lax.broadcasted_iota(jnp.int32, (num_ops, num_ops), 1)
    lt = (r0 <= c0).astype(jnp.float32)
    kidx = jax.lax.broadcasted_iota(jnp.int32, (M, num_jobs, num_ops), 1)
    ovals = jax.lax.broadcasted_iota(jnp.int32, (M, num_jobs, num_ops), 2)
    rows = jax.lax.broadcasted_iota(jnp.int32, (M + 1, P2), 0)
    conj_src = jnp.broadcast_to(conj_ref[0, 0:1, :], (M + 1, P2))
    conj_dst = jnp.broadcast_to(conj_ref[0, 1:2, :], (M + 1, P2))

    for i in range(bsz):
        mask = (pt_ref[i] > 0.0).astype(jnp.float32)
        csum = jax.lax.dot(mask, lt, precision=jax.lax.Precision.HIGHEST)
        rank = csum.astype(jnp.int32) - 1
        sel = jnp.where((rank[:, None, :] == kidx) & (mask[:, None, :] > 0.0),
                        ovals, 0)
        ops = jnp.sum(sel, axis=2).astype(jnp.float32)
        ops_pad = jnp.concatenate(
            [jnp.zeros((1, num_jobs), jnp.float32), ops], axis=0)
        dis_src = jax.lax.dot(ops_pad, psel_ref[0],
                              precision=jax.lax.Precision.HIGHEST)
        dis_dst = jax.lax.dot(ops_pad, psel_ref[1],
                              precision=jax.lax.Precision.HIGHEST)
        off = ((g * bsz + i) * num_ops).astype(jnp.int32)
        src = jnp.where(rows == 0, conj_src, dis_src.astype(jnp.int32)) + off
        dst = jnp.where(rows == 0, conj_dst, dis_dst.astype(jnp.int32)) + off
        for blk in range(M + 1):
            col = i * E + blk * P2
            ei_ref[0:1, pl.ds(col, P2)] = src[blk:blk + 1, :]
            ei_ref[1:2, pl.ds(col, P2)] = dst[blk:blk + 1, :]


def _x_copy_kernel(src_ref, dst_ref):
    dst_ref[:] = src_ref[:]


def _make_emb_sc(B, E, E_conj, D, T1R, inst_per_w, NC, L):
    E_dis = E - E_conj
    mesh = plsc.VectorSubcoreMesh(core_axis_name="c", subcore_axis_name="s")

    @functools.partial(
        pl.kernel,
        out_type=jax.ShapeDtypeStruct((B * E, D), jnp.float32),
        mesh=mesh,
        scratch_types=[
            pltpu.VMEM((2, D), jnp.float32),
            pltpu.VMEM((E_conj, D), jnp.float32),
            pltpu.VMEM((T1R, D), jnp.float32),
            pltpu.SemaphoreType.DMA,
        ],
        compiler_params=pltpu.CompilerParams(use_tc_tiling_on_sc=False),
    )
    def emb_sc(tab_hbm, out_hbm, tab_v, t0_v, t1_v, sem):
        wid = lax.axis_index("s") * NC + lax.axis_index("c")
        pltpu.sync_copy(tab_hbm, tab_v)
        row0 = [tab_v[0, pl.ds(c * L, L)] for c in range(D // L)]
        row1 = [tab_v[1, pl.ds(c * L, L)] for c in range(D // L)]

        def fill(r, _):
            for u in range(4):
                for c in range(D // L):
                    t0_v[r * 4 + u, pl.ds(c * L, L)] = row0[c]
                    t1_v[r * 4 + u, pl.ds(c * L, L)] = row1[c]
            return 0

        lax.fori_loop(0, E_conj // 4, fill, 0)
        rem = E_conj % 4
        for u in range(rem):
            for c in range(D // L):
                t0_v[E_conj - rem + u, pl.ds(c * L, L)] = row0[c]
                t1_v[E_conj - rem + u, pl.ds(c * L, L)] = row1[c]
        copies = []
        for ib in range(inst_per_w):
            base = (wid * inst_per_w + ib) * E
            copies.append(
                pltpu.async_copy(t0_v, out_hbm.at[pl.ds(base, E_conj)], sem))
            for j in range(E_dis // T1R):
                copies.append(pltpu.async_copy(
                    t1_v, out_hbm.at[pl.ds(base + E_conj + j * T1R, T1R)],
                    sem))
        for cp in copies:
            cp.wait()

    return emb_sc


def _kernel_impl(proc_times, init_embeddings, edge_embed_table):
    B, M, num_ops = proc_times.shape
    num_jobs = num_ops // M
    D = edge_embed_table.shape[1]
    E_conj = num_jobs * (M - 1)
    P2 = num_jobs * (num_jobs - 1)
    E = E_conj + M * P2
    BSZ = 32
    W = BSZ * E

    op_ids = np.arange(num_ops).reshape(num_jobs, M)
    conj = np.stack([op_ids[:, :-1].reshape(-1), op_ids[:, 1:].reshape(-1)],
                    axis=0).astype(np.int32)
    ii, jj = np.triu_indices(num_jobs, k=1)
    pat_src = np.concatenate([ii, jj])
    pat_dst = np.concatenate([jj, ii])
    psel = np.zeros((2, num_jobs, P2), dtype=np.float32)
    psel[0, pat_src, np.arange(P2)] = 1.0
    psel[1, pat_dst, np.arange(P2)] = 1.0
    conj3 = conj.reshape(1, 2, E_conj)

    kfn = functools.partial(_edge_index_kernel, BSZ, num_jobs, M, num_ops)
    edge_index = pl.pallas_call(
        kfn,
        grid=(B // BSZ,),
        in_specs=[
            pl.BlockSpec((1, 2, E_conj), lambda b: (0, 0, 0)),
            pl.BlockSpec((2, num_jobs, P2), lambda b: (0, 0, 0)),
            pl.BlockSpec((BSZ, M, num_ops), lambda b: (b, 0, 0)),
        ],
        out_specs=pl.BlockSpec((2, W), lambda b: (0, b)),
        out_shape=jax.ShapeDtypeStruct((2, B * E), jnp.int32),
    )(conj3, psel, proc_times)

    XB = 4
    xout = pl.pallas_call(
        _x_copy_kernel,
        grid=(B // XB,),
        in_specs=[pl.BlockSpec((XB, num_ops, D), lambda b: (b, 0, 0))],
        out_specs=pl.BlockSpec((XB, num_ops, D), lambda b: (b, 0, 0)),
        out_shape=jax.ShapeDtypeStruct((B, num_ops, D), jnp.float32),
    )(init_embeddings)

    info = plsc.get_sparse_core_info()
    NC, NS, L = info.num_cores, info.num_subcores, info.num_lanes
    NW = NC * NS
    inst_per_w = B // NW
    T1R = E_conj
    emb = _make_emb_sc(B, E, E_conj, D, T1R, inst_per_w, NC, L)(
        edge_embed_table)

    x = xout.reshape(-1, D)
    return x, edge_index, emb


@functools.lru_cache(maxsize=None)
def _jitted_kernel(device):
    sharding = jax.sharding.SingleDeviceSharding(device)

    def fmt(rank, tiling=None):
        return jax_layout.Format(
            jax_layout.Layout(major_to_minor=tuple(range(rank)),
                              tiling=tiling), sharding)

    return jax.jit(
        _kernel_impl,
        in_shardings=(fmt(3), fmt(3), fmt(2)),
        out_shardings=(fmt(2), fmt(2), fmt(2, tiling=((8,),))),
    )


def kernel(proc_times, init_embeddings, edge_embed_table):
    return _jitted_kernel(jax.devices()[0])(
        proc_times, init_embeddings, edge_embed_table)


# speedup vs baseline: 1.2873x; 1.2873x over previous
import functools

import jax
import jax.numpy as jnp
import numpy as np
from jax import lax
from jax.experimental import layout as jax_layout
from jax.experimental import pallas as pl
from jax.experimental.pallas import tpu as pltpu
from jax.experimental.pallas import tpu_sc as plsc


def _edge_index_kernel(bsz, num_jobs, M, num_ops, conj_ref, psel_ref, pt_ref,
                       ei_ref):
    g = pl.program_id(0)
    P2 = num_jobs * (num_jobs - 1)
    E = (M + 1) * P2

    r0 = jax.lax.broadcasted_iota(jnp.int32, (num_ops, num_ops), 0)
    c0 = jax.lax.broadcasted_iota(jnp.int32, (num_ops, num_ops), 1)
    lt = (r0 <= c0).astype(jnp.float32)
    kidx = jax.lax.broadcasted_iota(jnp.int32, (M, num_jobs, num_ops), 1)
    ovals = jax.lax.broadcasted_iota(jnp.int32, (M, num_jobs, num_ops), 2)
    rows = jax.lax.broadcasted_iota(jnp.int32, (M + 1, P2), 0)
    conj_src = jnp.broadcast_to(conj_ref[0, 0:1, :], (M + 1, P2))
    conj_dst = jnp.broadcast_to(conj_ref[0, 1:2, :], (M + 1, P2))

    for i in range(bsz):
        mask = (pt_ref[i] > 0.0).astype(jnp.float32)
        csum = jax.lax.dot(mask, lt, precision=jax.lax.Precision.HIGHEST)
        rank = csum.astype(jnp.int32) - 1
        sel = jnp.where((rank[:, None, :] == kidx) & (mask[:, None, :] > 0.0),
                        ovals, 0)
        ops = jnp.sum(sel, axis=2).astype(jnp.float32)
        ops_pad = jnp.concatenate(
            [jnp.zeros((1, num_jobs), jnp.float32), ops], axis=0)
        dis_src = jax.lax.dot(ops_pad, psel_ref[0],
                              precision=jax.lax.Precision.HIGHEST)
        dis_dst = jax.lax.dot(ops_pad, psel_ref[1],
                              precision=jax.lax.Precision.HIGHEST)
        off = ((g * bsz + i) * num_ops).astype(jnp.int32)
        src = jnp.where(rows == 0, conj_src, dis_src.astype(jnp.int32)) + off
        dst = jnp.where(rows == 0, conj_dst, dis_dst.astype(jnp.int32)) + off
        for blk in range(M + 1):
            col = i * E + blk * P2
            ei_ref[0:1, pl.ds(col, P2)] = src[blk:blk + 1, :]
            ei_ref[1:2, pl.ds(col, P2)] = dst[blk:blk + 1, :]


def _x_copy_kernel(src_ref, dst_ref):
    dst_ref[:] = src_ref[:]


def _make_emb_sc(B, E, E_conj, D, inst_per_w, NC, L):
    mesh = plsc.VectorSubcoreMesh(core_axis_name="c", subcore_axis_name="s")
    XR = E_conj + 8 + E_conj
    T1R = 200
    RA = E - E_conj - 8
    RB = E - E_conj

    @functools.partial(
        pl.kernel,
        out_type=jax.ShapeDtypeStruct((B * E, D), jnp.float32),
        mesh=mesh,
        scratch_types=[
            pltpu.VMEM((2, D), jnp.float32),
            pltpu.VMEM((XR, D), jnp.float32),
            pltpu.VMEM((T1R, D), jnp.float32),
            pltpu.SemaphoreType.DMA,
        ],
        compiler_params=pltpu.CompilerParams(use_tc_tiling_on_sc=True),
    )
    def emb_sc(tab_hbm, out_hbm, tab_v, x_v, t1_v, sem):
        wid = lax.axis_index("s") * NC + lax.axis_index("c")
        pltpu.sync_copy(tab_hbm, tab_v)
        row0 = [tab_v[0, pl.ds(c * L, L)] for c in range(D // L)]
        row1 = [tab_v[1, pl.ds(c * L, L)] for c in range(D // L)]

        def fillx(r, _):
            for u in range(2):
                for c in range(D // L):
                    x_v[r * 2 + u, pl.ds(c * L, L)] = row0[c]
            return 0

        lax.fori_loop(0, XR // 2, fillx, 0)
        for u in range(8):
            for c in range(D // L):
                x_v[E_conj + u, pl.ds(c * L, L)] = row1[c]

        def fill1(r, _):
            for u in range(2):
                for c in range(D // L):
                    t1_v[r * 2 + u, pl.ds(c * L, L)] = row1[c]
            return 0

        lax.fori_loop(0, T1R // 2, fill1, 0)

        r0 = wid * (inst_per_w * E)
        copies = []

        def t1_run(start, nrows):
            for j in range(nrows // T1R):
                copies.append(pltpu.async_copy(
                    t1_v, out_hbm.at[pl.ds(start + j * T1R, T1R)], sem))
            rem = nrows % T1R
            if rem:
                copies.append(pltpu.async_copy(
                    t1_v.at[pl.ds(0, rem)],
                    out_hbm.at[pl.ds(start + nrows - rem, rem)], sem))

        copies.append(pltpu.async_copy(
            x_v.at[pl.ds(0, E_conj + 4)],
            out_hbm.at[pl.ds(r0, E_conj + 4)], sem))
        t1_run(r0 + E_conj + 4, RA)
        copies.append(pltpu.async_copy(
            x_v.at[pl.ds(E_conj + 4, E_conj + 4)],
            out_hbm.at[pl.ds(r0 + E - 4, E_conj + 4)], sem))
        t1_run(r0 + E + E_conj, RB)
        for cp in copies:
            cp.wait()

    return emb_sc


def _kernel_impl(proc_times, init_embeddings, edge_embed_table):
    B, M, num_ops = proc_times.shape
    num_jobs = num_ops // M
    D = edge_embed_table.shape[1]
    E_conj = num_jobs * (M - 1)
    P2 = num_jobs * (num_jobs - 1)
    E = E_conj + M * P2
    BSZ = 32
    W = BSZ * E

    op_ids = np.arange(num_ops).reshape(num_jobs, M)
    conj = np.stack([op_ids[:, :-1].reshape(-1), op_ids[:, 1:].reshape(-1)],
                    axis=0).astype(np.int32)
    ii, jj = np.triu_indices(num_jobs, k=1)
    pat_src = np.concatenate([ii, jj])
    pat_dst = np.concatenate([jj, ii])
    psel = np.zeros((2, num_jobs, P2), dtype=np.float32)
    psel[0, pat_src, np.arange(P2)] = 1.0
    psel[1, pat_dst, np.arange(P2)] = 1.0
    conj3 = conj.reshape(1, 2, E_conj)

    kfn = functools.partial(_edge_index_kernel, BSZ, num_jobs, M, num_ops)
    edge_index = pl.pallas_call(
        kfn,
        grid=(B // BSZ,),
        in_specs=[
            pl.BlockSpec((1, 2, E_conj), lambda b: (0, 0, 0)),
            pl.BlockSpec((2, num_jobs, P2), lambda b: (0, 0, 0)),
            pl.BlockSpec((BSZ, M, num_ops), lambda b: (b, 0, 0)),
        ],
        out_specs=pl.BlockSpec((2, W), lambda b: (0, b)),
        out_shape=jax.ShapeDtypeStruct((2, B * E), jnp.int32),
    )(conj3, psel, proc_times)

    XB = 4
    xout = pl.pallas_call(
        _x_copy_kernel,
        grid=(B // XB,),
        in_specs=[pl.BlockSpec((XB, num_ops, D), lambda b: (b, 0, 0))],
        out_specs=pl.BlockSpec((XB, num_ops, D), lambda b: (b, 0, 0)),
        out_shape=jax.ShapeDtypeStruct((B, num_ops, D), jnp.float32),
    )(init_embeddings)

    info = plsc.get_sparse_core_info()
    NC, NS, L = info.num_cores, info.num_subcores, info.num_lanes
    NW = NC * NS
    inst_per_w = B // NW
    emb = _make_emb_sc(B, E, E_conj, D, inst_per_w, NC, L)(edge_embed_table)

    x = xout.reshape(-1, D)
    return x, edge_index, emb


@functools.lru_cache(maxsize=None)
def _jitted_kernel(device):
    sharding = jax.sharding.SingleDeviceSharding(device)

    def fmt(rank, tiling=None):
        return jax_layout.Format(
            jax_layout.Layout(major_to_minor=tuple(range(rank)),
                              tiling=tiling), sharding)

    return jax.jit(
        _kernel_impl,
        in_shardings=(fmt(3), fmt(3), fmt(2)),
        out_shardings=(fmt(2), fmt(2), fmt(2)),
    )


def kernel(proc_times, init_embeddings, edge_embed_table):
    return _jitted_kernel(jax.devices()[0])(
        proc_times, init_embeddings, edge_embed_table)
